# Initial kernel scaffold; baseline (speedup 1.0000x reference)
import jax
import jax.numpy as jnp
from jax.experimental import pallas as pl


def kernel(x, table):
    raise NotImplementedError("write your pallas kernel here")


# speedup vs baseline: 3.2076x; 3.2076x over previous
import functools

import jax
import jax.numpy as jnp
from jax import lax
from jax.experimental import pallas as pl
from jax.experimental.pallas import tpu as pltpu
from jax.experimental.pallas import tpu_sc as plsc

VOCAB = 100000
DIM = 64
BATCH = 4096
HIST = 50

B = BATCH * HIST
NC, NS = 2, 16
NW = NC * NS
BPW = B // NW
CHUNK = 800
STEPS = BPW // CHUNK
SCALE = 8.0

_mesh = plsc.VectorSubcoreMesh(core_axis_name="c", subcore_axis_name="s")


@functools.partial(
    pl.kernel,
    out_type=jax.ShapeDtypeStruct((B, DIM), jnp.float32),
    mesh=_mesh,
    scratch_types=[
        pltpu.VMEM((CHUNK,), jnp.int32),
        pltpu.VMEM((CHUNK, DIM), jnp.float32),
        pltpu.SemaphoreType.DMA,
    ],
    compiler_params=pltpu.CompilerParams(use_tc_tiling_on_sc=False),
)
def _emb_lookup(x_hbm, table_hbm, out_hbm, idx_v, rows_v, sem):
    wid = lax.axis_index("s") * NC + lax.axis_index("c")
    base = wid * BPW

    def step(s, carry):
        off = base + s * CHUNK
        pltpu.sync_copy(x_hbm.at[pl.ds(off, CHUNK)], idx_v)
        pltpu.async_copy(table_hbm.at[idx_v], rows_v, sem).wait()

        def row(r, c):
            for j in range(DIM // 16):
                sl = pl.ds(j * 16, 16)
                rows_v[r, sl] = rows_v[r, sl] * SCALE
            return c

        lax.fori_loop(0, CHUNK, row, 0)
        pltpu.sync_copy(rows_v, out_hbm.at[pl.ds(off, CHUNK)])
        return carry

    lax.fori_loop(0, STEPS, step, 0)


def kernel(x, table):
    out = _emb_lookup(x.reshape(-1), table)
    return out.reshape(BATCH, HIST, DIM)


# speedup vs baseline: 3.6067x; 1.1244x over previous
import functools

import jax
import jax.numpy as jnp
from jax import lax
from jax.experimental import pallas as pl
from jax.experimental.pallas import tpu as pltpu
from jax.experimental.pallas import tpu_sc as plsc

VOCAB = 100000
DIM = 64
BATCH = 4096
HIST = 50

B = BATCH * HIST
NC, NS = 2, 16
NW = NC * NS
BPW = B // NW
CHUNK = 800
STEPS = BPW // CHUNK
SCALE = 8.0

_mesh = plsc.VectorSubcoreMesh(core_axis_name="c", subcore_axis_name="s")


@functools.partial(
    pl.kernel,
    out_type=jax.ShapeDtypeStruct((B, DIM), jnp.float32),
    mesh=_mesh,
    scratch_types=[
        pltpu.VMEM((CHUNK,), jnp.int32),
        pltpu.VMEM((CHUNK,), jnp.int32),
        pltpu.VMEM((CHUNK, DIM), jnp.float32),
        pltpu.VMEM((CHUNK, DIM), jnp.float32),
        pltpu.SemaphoreType.DMA,
        pltpu.SemaphoreType.DMA,
        pltpu.SemaphoreType.DMA,
        pltpu.SemaphoreType.DMA,
    ],
    compiler_params=pltpu.CompilerParams(use_tc_tiling_on_sc=False),
)
def _emb_lookup(x_hbm, table_hbm, out_hbm, idx0, idx1, rows0, rows1,
                gs0, gs1, os0, os1):
    wid = lax.axis_index("s") * NC + lax.axis_index("c")
    base = wid * BPW
    idx = (idx0, idx1)
    rows = (rows0, rows1)
    gsem = (gs0, gs1)
    osem = (os0, os1)

    def start_gather(s):
        b = s % 2
        off = base + s * CHUNK
        pltpu.sync_copy(x_hbm.at[pl.ds(off, CHUNK)], idx[b])
        return pltpu.async_copy(table_hbm.at[idx[b]], rows[b], gsem[b])

    gathers = [None] * STEPS
    writes = [None] * STEPS
    gathers[0] = start_gather(0)
    for s in range(STEPS):
        b = s % 2
        if s + 1 < STEPS:
            if s >= 1:
                writes[s - 1].wait()
            gathers[s + 1] = start_gather(s + 1)
        gathers[s].wait()

        def row(r, c):
            for rr in range(4):
                for j in range(DIM // 16):
                    sl = pl.ds(j * 16, 16)
                    rows[b][r * 4 + rr, sl] = rows[b][r * 4 + rr, sl] * SCALE
            return c

        lax.fori_loop(0, CHUNK // 4, row, 0)
        writes[s] = pltpu.async_copy(
            rows[b], out_hbm.at[pl.ds(base + s * CHUNK, CHUNK)], osem[b])
    writes[STEPS - 2].wait()
    writes[STEPS - 1].wait()


def kernel(x, table):
    out = _emb_lookup(x.reshape(-1), table)
    return out.reshape(BATCH, HIST, DIM)
